# baseline (speedup 1.0000x reference)
import jax
import jax.numpy as jnp
from jax import lax
from jax.experimental import pallas as pl
from jax.experimental.pallas import tpu as pltpu
from jax.experimental.pallas import tpu_sc as plsc

N = 50000
E = 1600000
IN = 128
H = 3
C1 = 15
C2 = 9
L1 = 100
L2 = 4
NA = 50
NB_MOL = N // NA
OUT = 128

NP = 51200
SL = NP // 16
PW = 51200
EP = 32 * PW
NROWS = EP // 128
RW = PW // 128
OUTER = 25
INNER = 16

BN_BLK = 2000
NBLK = N // BN_BLK


def _make_sc_head(with_deg):
    mesh = plsc.VectorSubcoreMesh(core_axis_name="c", subcore_axis_name="s")
    out_type = [jax.ShapeDtypeStruct((2, NP, 16), jnp.float32)]
    if with_deg:
        out_type.append(jax.ShapeDtypeStruct((2 * NP,), jnp.float32))
    scratch = [
        pltpu.VMEM((INNER, 128), jnp.int32),
        pltpu.VMEM((INNER, 128), jnp.int32),
        pltpu.VMEM((128,), jnp.float32),
        pltpu.VMEM((128,), jnp.float32),
        pltpu.VMEM((128, 16), jnp.float32),
        pltpu.VMEM((128,), jnp.float32),
        pltpu.VMEM((16,), jnp.float32),
        pltpu.VMEM((128, 16), jnp.float32),
        pltpu.VMEM((640,), jnp.float32),
        pltpu.VMEM_SHARED((NP, 16), jnp.float32),
    ]
    if with_deg:
        scratch.append(pltpu.VMEM_SHARED((NP,), jnp.float32))

    def body(src_hbm, dst_hbm, xtab_hbm, sa_hbm, da_hbm, as_hbm, *rest):
        if with_deg:
            (s_out, deg_out, src_v, dst_v, asv, adv, xrow, onesb, as_vm,
             z2d, z1d, acc_s, acc_deg) = rest
        else:
            (s_out, src_v, dst_v, asv, adv, xrow, onesb, as_vm,
             z2d, z1d, acc_s) = rest
        c = lax.axis_index("c")
        s = lax.axis_index("s")
        wid = c * 16 + s

        sl0 = pl.multiple_of(s * SL, 128)
        zero16 = jnp.zeros((16,), jnp.float32)
        for t in range(128):
            z2d[pl.ds(t, 1), :] = zero16.reshape(1, 16)
        for t in range(40):
            z1d[pl.ds(t * 16, 16)] = zero16
        for t in range(8):
            onesb[pl.ds(t * 16, 16)] = zero16 + 1.0
        for t in range(25):
            pltpu.sync_copy(z2d, acc_s.at[pl.ds(sl0 + t * 128, 128), :])
        if with_deg:
            for t in range(5):
                pltpu.sync_copy(z1d, acc_deg.at[pl.ds(sl0 + t * 640, 640)])
        pltpu.sync_copy(as_hbm, as_vm)
        plsc.subcore_barrier()

        as_v = as_vm[...]
        iota16 = lax.iota(jnp.int32, 16)
        base_row = wid * RW

        def inner(j, g):
            sr = src_v.at[j]
            dr = dst_v.at[j]
            pltpu.sync_copy(sa_hbm.at[sr], asv)
            pltpu.sync_copy(da_hbm.at[dr], adv)
            pltpu.sync_copy(xtab_hbm.at[sr], xrow)
            for k in range(8):
                sa = asv[pl.ds(k * 16, 16)]
                da = adv[pl.ds(k * 16, 16)]
                z = sa + da
                al = jnp.maximum(z, 0.2 * z)
                zm = as_v + da
                m = jnp.maximum(zm, 0.2 * zm)
                p = jnp.exp(al - m)
                dnums = lax.GatherDimensionNumbers(
                    offset_dims=(), collapsed_slice_dims=(0,),
                    start_index_map=(0,))
                for r16 in range(16):
                    r = k * 16 + r16
                    idx = jnp.full((16, 1), r16, jnp.int32)
                    pb = lax.gather(
                        p, idx, dnums, (1,),
                        mode=lax.GatherScatterMode.PROMISE_IN_BOUNDS)
                    xrow[r, :] = xrow[r, :] * pb
            pltpu.sync_copy(xrow, acc_s.at[dr], add=True)
            if with_deg:
                pltpu.sync_copy(onesb, acc_deg.at[dr], add=True)
            return g

        def outer(g, _):
            r0 = pl.multiple_of(base_row + g * INNER, 8)
            pltpu.sync_copy(src_hbm.at[pl.ds(r0, INNER), :], src_v)
            pltpu.sync_copy(dst_hbm.at[pl.ds(r0, INNER), :], dst_v)
            lax.fori_loop(0, INNER, inner, g)
            return 0

        lax.fori_loop(0, OUTER, outer, 0)
        plsc.subcore_barrier()
        pltpu.sync_copy(acc_s.at[pl.ds(sl0, SL), :],
                        s_out.at[c, pl.ds(sl0, SL), :])
        if with_deg:
            doff = pl.multiple_of(c * NP + s * SL, 128)
            pltpu.sync_copy(acc_deg.at[pl.ds(sl0, SL)],
                            deg_out.at[pl.ds(doff, SL)])

    return pl.kernel(body, out_type=tuple(out_type), mesh=mesh,
                     compiler_params=pltpu.CompilerParams(
                         use_tc_tiling_on_sc=False),
                     scratch_types=tuple(scratch))


def _sc_gcn_body(src_hbm, dst_hbm, ytab_hbm, g_out,
                 src_v, dst_v, xrow, z2d, acc_g):
    c = lax.axis_index("c")
    s = lax.axis_index("s")
    wid = c * 16 + s
    sl0 = pl.multiple_of(s * SL, 128)
    zero16 = jnp.zeros((16,), jnp.float32)
    for t in range(128):
        z2d[pl.ds(t, 1), :] = zero16.reshape(1, 16)
    for t in range(25):
        pltpu.sync_copy(z2d, acc_g.at[pl.ds(sl0 + t * 128, 128), :])
    plsc.subcore_barrier()

    base_row = wid * RW

    def inner(j, g):
        sr = src_v.at[j]
        dr = dst_v.at[j]
        pltpu.sync_copy(ytab_hbm.at[sr], xrow)
        pltpu.sync_copy(xrow, acc_g.at[dr], add=True)
        return g

    def outer(g, _):
        r0 = pl.multiple_of(base_row + g * INNER, 8)
        pltpu.sync_copy(src_hbm.at[pl.ds(r0, INNER), :], src_v)
        pltpu.sync_copy(dst_hbm.at[pl.ds(r0, INNER), :], dst_v)
        lax.fori_loop(0, INNER, inner, g)
        return 0

    lax.fori_loop(0, OUTER, outer, 0)
    plsc.subcore_barrier()
    pltpu.sync_copy(acc_g.at[pl.ds(sl0, SL), :],
                    g_out.at[c, pl.ds(sl0, SL), :])


def _make_sc_gcn():
    mesh = plsc.VectorSubcoreMesh(core_axis_name="c", subcore_axis_name="s")
    return pl.kernel(
        _sc_gcn_body,
        out_type=jax.ShapeDtypeStruct((2, NP, 16), jnp.float32),
        mesh=mesh,
        compiler_params=pltpu.CompilerParams(use_tc_tiling_on_sc=False),
        scratch_types=(
            pltpu.VMEM((INNER, 128), jnp.int32),
            pltpu.VMEM((INNER, 128), jnp.int32),
            pltpu.VMEM((128, 16), jnp.float32),
            pltpu.VMEM((128, 16), jnp.float32),
            pltpu.VMEM_SHARED((NP, 16), jnp.float32),
        ),
    )


def _tc1_body(x_ref, wext_ref, asd_ref, o0, o1, o2, aux_ref, mx_ref):
    xb = x_ref[...]
    xp48 = jax.lax.dot_general(xb, wext_ref[...], (((1,), (0,)), ((), ())),
                               preferred_element_type=jnp.float32)
    lane = jax.lax.broadcasted_iota(jnp.int32, (BN_BLK, 16), 1)
    cols = []
    for h, oref in enumerate((o0, o1, o2)):
        blk = xp48[:, 16 * h:16 * h + 16]
        oref[...] = jnp.where(lane == 15, 1.0, blk)
        a_s = jnp.sum(blk * asd_ref[0:1, 16 * h:16 * h + 16], axis=1,
                      keepdims=True)
        a_d = jnp.sum(blk * asd_ref[1:2, 16 * h:16 * h + 16], axis=1,
                      keepdims=True)
        cols.append((a_s, a_d))
    zpad = jnp.zeros((BN_BLK, 5), jnp.float32)
    auxb = jnp.concatenate(
        [cols[0][1], cols[1][1], cols[2][1], zpad,
         cols[0][0], cols[1][0], cols[2][0], zpad], axis=1)
    aux_ref[...] = auxb
    mx = jnp.max(auxb, axis=0, keepdims=True)
    mx_ref[...] = jnp.broadcast_to(mx, (8, 16))


def _tc2a_body(s0_ref, s1_ref, s2_ref, x0_ref, x1_ref, x2_ref, aux_ref,
               asrow_ref, b1p_ref, whp_ref, bhp_ref, hw_ref, st_ref):
    aux = aux_ref[...]
    hs = []
    for h, (s_ref, x_ref) in enumerate(((s0_ref, x0_ref), (s1_ref, x1_ref),
                                        (s2_ref, x2_ref))):
        ssum = s_ref[0] + s_ref[1]
        a_s = aux[:, 8 + h:9 + h]
        a_d = aux[:, h:h + 1]
        z = a_s + a_d
        al = jnp.maximum(z, 0.2 * z)
        zm = asrow_ref[0, 8 + h] + a_d
        m = jnp.maximum(zm, 0.2 * zm)
        ps = jnp.exp(al - m)
        numer = ssum + ps * x_ref[...]
        denom = ssum[:, 15:16] + ps + 1e-16
        hs.append(numer / denom)
    hcat = jnp.concatenate(hs, axis=1) + b1p_ref[0:1, :]
    hw = jax.lax.dot_general(hcat, whp_ref[...], (((1,), (0,)), ((), ())),
                             preferred_element_type=jnp.float32)
    hw = hw + bhp_ref[0:1, :]
    hw_ref[...] = hw

    @pl.when(pl.program_id(0) == 0)
    def _():
        st_ref[...] = jnp.zeros((8, 16), jnp.float32)

    s1 = jnp.sum(hw, axis=0, keepdims=True)
    s2 = jnp.sum(hw * hw, axis=0, keepdims=True)
    upd = jnp.concatenate([s1, s2, jnp.zeros((6, 16), jnp.float32)], axis=0)
    st_ref[...] = st_ref[...] + upd


def _tc2b_body(hw_ref, dinvb_ref, w2p_ref, m1_ref, isg1_ref, be1_ref,
               c2_ref, y_ref):
    hn = (hw_ref[...] - m1_ref[0:1, :]) * isg1_ref[0:1, :] + be1_ref[0:1, :]
    t = jax.lax.dot_general(hn, w2p_ref[...], (((1,), (0,)), ((), ())),
                            preferred_element_type=jnp.float32)
    y_ref[...] = (t + c2_ref[0:1, :]) * dinvb_ref[...]


def _tc3a_body(g_ref, y_ref, b2p_ref, h2_ref, st_ref):
    y = y_ref[...]
    gsum = g_ref[0] + g_ref[1]
    dinv = y[:, 9:10]
    t = dinv * (gsum + y) + b2p_ref[0:1, :]
    h2_ref[...] = t

    @pl.when(pl.program_id(0) == 0)
    def _():
        st_ref[...] = jnp.zeros((8, 16), jnp.float32)

    s1 = jnp.sum(t, axis=0, keepdims=True)
    s2 = jnp.sum(t * t, axis=0, keepdims=True)
    upd = jnp.concatenate([s1, s2, jnp.zeros((6, 16), jnp.float32)], axis=0)
    st_ref[...] = st_ref[...] + upd


def _tc3b_body(h2_ref, m2_ref, isg2_ref, be2_ref, a1_ref, bb1_ref, a2_ref,
               bl2_ref, z2_ref):
    hn = (h2_ref[...] - m2_ref[0:1, :]) * isg2_ref[0:1, :] + be2_ref[0:1, :]
    z1 = jax.lax.dot_general(hn, a1_ref[...], (((1,), (0,)), ((), ())),
                             preferred_element_type=jnp.float32)
    z1 = z1 + bb1_ref[0:1, :]
    z1 = jnp.maximum(z1, 0.01 * z1)
    z2 = jax.lax.dot_general(z1, a2_ref[...], (((1,), (0,)), ((), ())),
                             preferred_element_type=jnp.float32)
    z2 = z2 + bl2_ref[0:1, :]
    z2_ref[...] = jnp.maximum(z2, 0.01 * z2)


def _tc3c_body(hr_ref, wt_ref, btp_ref, wmu_ref, bmup_ref, wls_ref, blsp_ref,
               mu_ref, ls_ref):
    z3 = jax.lax.dot_general(hr_ref[...], wt_ref[...], (((1,), (0,)), ((), ())),
                             preferred_element_type=jnp.float32)
    z3 = z3 + btp_ref[0:1, :]
    z3 = jnp.maximum(z3, 0.01 * z3)
    mu_ref[...] = jax.lax.dot_general(
        z3, wmu_ref[...], (((1,), (0,)), ((), ())),
        preferred_element_type=jnp.float32) + bmup_ref[0:1, :]
    ls_ref[...] = jax.lax.dot_general(
        z3, wls_ref[...], (((1,), (0,)), ((), ())),
        preferred_element_type=jnp.float32) + blsp_ref[0:1, :]


def _row8(v, width):
    out = jnp.zeros((8, width), jnp.float32)
    return out.at[0, :v.shape[0]].set(v)


def kernel(x, edge_index, W1, a_src, a_dst, b1, Wh, bh, g1, be1, W2, b2, g2,
           be2, Wl1, bl1, Wl2, bl2, Wt, bt, Wmu, bmu, Wls, bls):
    f32 = jnp.float32
    src = edge_index[0]
    dst = edge_index[1]
    padn = EP - E
    srcp = jnp.concatenate([src, jnp.full((padn,), N, jnp.int32)])
    dstp = jnp.concatenate([dst, jnp.full((padn,), N, jnp.int32)])
    src2 = srcp.reshape(NROWS, 128)
    dst2 = dstp.reshape(NROWS, 128)

    W1r = W1.reshape(IN, H, C1)
    Wext = jnp.concatenate([W1r, jnp.zeros((IN, H, 1), f32)],
                           axis=2).reshape(IN, 48)
    apad = jnp.zeros((H, 1), f32)
    asd = jnp.zeros((8, 48), f32)
    asd = asd.at[0].set(jnp.concatenate([a_src, apad], 1).reshape(48))
    asd = asd.at[1].set(jnp.concatenate([a_dst, apad], 1).reshape(48))

    grid25 = (NBLK,)
    bs_x = pl.BlockSpec((BN_BLK, IN), lambda i: (i, 0))
    bs_full = lambda shape: pl.BlockSpec(shape, lambda i: tuple(0 for _ in shape))
    bs_n16 = pl.BlockSpec((BN_BLK, 16), lambda i: (i, 0))
    xpad0, xpad1, xpad2, aux, mx = pl.pallas_call(
        _tc1_body,
        grid=grid25,
        in_specs=[bs_x, bs_full((IN, 48)), bs_full((8, 48))],
        out_specs=[bs_n16, bs_n16, bs_n16, bs_n16,
                   pl.BlockSpec((8, 16), lambda i: (i, 0))],
        out_shape=[jax.ShapeDtypeStruct((N, 16), f32)] * 4 +
                  [jax.ShapeDtypeStruct((8 * NBLK, 16), f32)],
    )(x, Wext, asd)

    As3 = jnp.max(mx[:, 8:11], axis=0)
    as16 = [jnp.full((16,), As3[h], f32) for h in range(H)]

    rowpad = jnp.zeros((NP - N, 16), f32)
    wordpad = jnp.zeros((NP - N,), f32)
    xtabs = [jnp.concatenate([xp, rowpad], axis=0)
             for xp in (xpad0, xpad1, xpad2)]
    sas = [jnp.concatenate([aux[:, 8 + h], wordpad]) for h in range(H)]
    das = [jnp.concatenate([aux[:, h], wordpad]) for h in range(H)]

    head0 = _make_sc_head(True)
    headx = _make_sc_head(False)
    S0, degp = head0(src2, dst2, xtabs[0], sas[0], das[0], as16[0])
    (S1,) = headx(src2, dst2, xtabs[1], sas[1], das[1], as16[1])
    (S2,) = headx(src2, dst2, xtabs[2], sas[2], das[2], as16[2])

    degp = degp.reshape(2, NP)
    deg = degp[0, :N] + degp[1, :N] + 1.0
    dinv = 1.0 / jnp.sqrt(jnp.maximum(deg, 1e-12))
    dinvb = jnp.broadcast_to(dinv[:, None], (N, 16))

    Whr = Wh.reshape(H, C1, C1)
    Whp = jnp.concatenate([Whr, jnp.zeros((H, 1, C1), f32)],
                          axis=1).reshape(48, C1)
    Whp = jnp.concatenate([Whp, jnp.zeros((48, 1), f32)], axis=1)
    b1r = b1.reshape(H, C1)
    b1p48 = jnp.concatenate([b1r, jnp.zeros((H, 1), f32)], axis=1).reshape(48)
    asrow = _row8(jnp.concatenate([jnp.zeros((8,), f32), As3]), 16)

    bs_s = pl.BlockSpec((2, BN_BLK, 16), lambda i: (0, i, 0))
    hw, st1 = pl.pallas_call(
        _tc2a_body,
        grid=grid25,
        in_specs=[bs_s, bs_s, bs_s, bs_n16, bs_n16, bs_n16, bs_n16,
                  bs_full((8, 16)), bs_full((8, 48)), bs_full((48, 16)),
                  bs_full((8, 16))],
        out_specs=[bs_n16, pl.BlockSpec((8, 16), lambda i: (0, 0))],
        out_shape=[jax.ShapeDtypeStruct((N, 16), f32),
                   jax.ShapeDtypeStruct((8, 16), f32)],
    )(S0, S1, S2, xpad0, xpad1, xpad2, aux, asrow,
      _row8(b1p48, 48), Whp, _row8(bh, 16))

    mean1 = st1[0] / N
    var1 = st1[1] / N - mean1 * mean1
    g1p = jnp.concatenate([g1, jnp.zeros((1,), f32)])
    be1p = jnp.concatenate([be1, jnp.zeros((1,), f32)])
    isg1 = g1p / jnp.sqrt(var1 + 1e-5)
    W2p16 = jnp.zeros((16, 16), f32).at[:C1, :C2].set(W2)
    c2 = jnp.zeros((16,), f32).at[9].set(1.0)

    (y,) = pl.pallas_call(
        _tc2b_body,
        grid=grid25,
        in_specs=[bs_n16, bs_n16, bs_full((16, 16)), bs_full((8, 16)),
                  bs_full((8, 16)), bs_full((8, 16)), bs_full((8, 16))],
        out_specs=[bs_n16],
        out_shape=[jax.ShapeDtypeStruct((N, 16), f32)],
    )(hw, dinvb, W2p16, _row8(mean1, 16), _row8(isg1, 16), _row8(be1p, 16),
      _row8(c2, 16))

    ytab = jnp.concatenate([y, rowpad], axis=0)

    gcn = _make_sc_gcn()
    Gp = gcn(src2, dst2, ytab)

    h2, st2 = pl.pallas_call(
        _tc3a_body,
        grid=grid25,
        in_specs=[bs_s, bs_n16, bs_full((8, 16))],
        out_specs=[bs_n16, pl.BlockSpec((8, 16), lambda i: (0, 0))],
        out_shape=[jax.ShapeDtypeStruct((N, 16), f32),
                   jax.ShapeDtypeStruct((8, 16), f32)],
    )(Gp, y, _row8(b2, 16))

    mean2 = st2[0] / N
    var2 = st2[1] / N - mean2 * mean2
    g2p = jnp.zeros((16,), f32).at[:C2].set(g2)
    be2p = jnp.zeros((16,), f32).at[:C2].set(be2)
    isg2 = g2p / jnp.sqrt(var2 + 1e-5)

    A1p = jnp.zeros((16, 128), f32).at[:C2, :L1].set(Wl1)
    bb1 = jnp.zeros((128,), f32).at[:L1].set(bl1)
    A2p = jnp.zeros((128, 8), f32).at[:L1, :L2].set(Wl2)
    bl2p = jnp.zeros((8,), f32).at[:L2].set(bl2)

    (z2,) = pl.pallas_call(
        _tc3b_body,
        grid=grid25,
        in_specs=[bs_n16, bs_full((8, 16)), bs_full((8, 16)),
                  bs_full((8, 16)), bs_full((16, 128)), bs_full((8, 128)),
                  bs_full((128, 8)), bs_full((8, 8))],
        out_specs=[pl.BlockSpec((BN_BLK, 8), lambda i: (i, 0))],
        out_shape=[jax.ShapeDtypeStruct((N, 8), f32)],
    )(h2, _row8(mean2, 16), _row8(isg2, 16), _row8(be2p, 16), A1p,
      _row8(bb1, 128), A2p, _row8(bl2p, 8))

    hresh = z2[:, :L2].reshape(NB_MOL, NA * L2)

    mu, ls = pl.pallas_call(
        _tc3c_body,
        grid=(1,),
        in_specs=[bs_full((NB_MOL, NA * L2)), bs_full((NA * L2, OUT)),
                  bs_full((8, OUT)), bs_full((OUT, OUT)), bs_full((8, OUT)),
                  bs_full((OUT, OUT)), bs_full((8, OUT))],
        out_specs=[bs_full((NB_MOL, OUT)), bs_full((NB_MOL, OUT))],
        out_shape=[jax.ShapeDtypeStruct((NB_MOL, OUT), f32),
                   jax.ShapeDtypeStruct((NB_MOL, OUT), f32)],
    )(hresh, Wt, _row8(bt, OUT), Wmu, _row8(bmu, OUT), Wls, _row8(bls, OUT))

    return (mu, ls, edge_index)


# speedup vs baseline: 2.0806x; 2.0806x over previous
import jax
import jax.numpy as jnp
from jax import lax
from jax.experimental import pallas as pl
from jax.experimental.pallas import tpu as pltpu
from jax.experimental.pallas import tpu_sc as plsc

N = 50000
E = 1600000
IN = 128
H = 3
C1 = 15
C2 = 9
L1 = 100
L2 = 4
NA = 50
NB_MOL = N // NA
OUT = 128

NP = 51200
SL = NP // 16
PW = 51200
EP = 32 * PW
NROWS = EP // 128
RW = PW // 128
OUTER = 25
INNER = 16

BN_BLK = 2000
NBLK = N // BN_BLK


def _make_sc_head(with_deg):
    mesh = plsc.VectorSubcoreMesh(core_axis_name="c", subcore_axis_name="s")
    out_type = [jax.ShapeDtypeStruct((2, NP, 16), jnp.float32)]
    if with_deg:
        out_type.append(jax.ShapeDtypeStruct((2 * NP,), jnp.float32))
    scratch = [
        pltpu.VMEM((INNER, 2, 128), jnp.int32),
        pltpu.VMEM((128,), jnp.float32),
        pltpu.VMEM((128,), jnp.float32),
        pltpu.VMEM((128,), jnp.float32),
        pltpu.VMEM((128,), jnp.float32),
        pltpu.VMEM((128, 16), jnp.float32),
        pltpu.VMEM((128, 16), jnp.float32),
        pltpu.VMEM((128,), jnp.float32),
        pltpu.VMEM((16,), jnp.float32),
        pltpu.VMEM((128, 16), jnp.float32),
        pltpu.VMEM((640,), jnp.float32),
        pltpu.SemaphoreType.DMA,
        pltpu.SemaphoreType.DMA,
        pltpu.VMEM_SHARED((NP, 16), jnp.float32),
    ]
    if with_deg:
        scratch.append(pltpu.VMEM_SHARED((NP,), jnp.float32))

    def body(ei_hbm, xtab_hbm, sa_hbm, da_hbm, as_hbm, *rest):
        if with_deg:
            (s_out, deg_out, ev, asvA, asvB, advA, advB, xrowA, xrowB,
             onesb, as_vm, z2d, z1d, gsemA, gsemB, acc_s, acc_deg) = rest
        else:
            (s_out, ev, asvA, asvB, advA, advB, xrowA, xrowB,
             onesb, as_vm, z2d, z1d, gsemA, gsemB, acc_s) = rest
        c = lax.axis_index("c")
        s = lax.axis_index("s")
        wid = c * 16 + s

        sl0 = pl.multiple_of(s * SL, 128)
        zero16 = jnp.zeros((16,), jnp.float32)
        for t in range(128):
            z2d[pl.ds(t, 1), :] = zero16.reshape(1, 16)
        for t in range(40):
            z1d[pl.ds(t * 16, 16)] = zero16
        for t in range(8):
            onesb[pl.ds(t * 16, 16)] = zero16 + 1.0
        for t in range(25):
            pltpu.sync_copy(z2d, acc_s.at[pl.ds(sl0 + t * 128, 128), :])
        if with_deg:
            for t in range(5):
                pltpu.sync_copy(z1d, acc_deg.at[pl.ds(sl0 + t * 640, 640)])
        pltpu.sync_copy(as_hbm, as_vm)
        plsc.subcore_barrier()

        as_v = as_vm[...]
        base_row = wid * RW

        slots = ((asvA, advA, xrowA, gsemA), (asvB, advB, xrowB, gsemB))

        def start_g(slot, j):
            asv, adv, xrow, gsem = slot
            sr = ev.at[j, 0]
            dr = ev.at[j, 1]
            pltpu.async_copy(sa_hbm.at[sr], asv, gsem)
            pltpu.async_copy(da_hbm.at[dr], adv, gsem)
            pltpu.async_copy(xtab_hbm.at[sr], xrow, gsem)

        def finish(slot, j):
            asv, adv, xrow, gsem = slot
            sr = ev.at[j, 0]
            dr = ev.at[j, 1]
            pltpu.make_async_copy(sa_hbm.at[sr], asv, gsem).wait()
            pltpu.make_async_copy(da_hbm.at[dr], adv, gsem).wait()
            pltpu.make_async_copy(xtab_hbm.at[sr], xrow, gsem).wait()
            dnums = lax.GatherDimensionNumbers(
                offset_dims=(), collapsed_slice_dims=(0,),
                start_index_map=(0,))
            for k in range(8):
                sa = asv[pl.ds(k * 16, 16)]
                da = adv[pl.ds(k * 16, 16)]
                z = sa + da
                al = jnp.maximum(z, 0.2 * z)
                zm = as_v + da
                m = jnp.maximum(zm, 0.2 * zm)
                p = jnp.exp(al - m)
                for r16 in range(16):
                    r = k * 16 + r16
                    idx = jnp.full((16, 1), r16, jnp.int32)
                    pb = lax.gather(
                        p, idx, dnums, (1,),
                        mode=lax.GatherScatterMode.PROMISE_IN_BOUNDS)
                    xrow[r, :] = xrow[r, :] * pb
            pltpu.sync_copy(xrow, acc_s.at[dr], add=True)
            if with_deg:
                pltpu.sync_copy(onesb, acc_deg.at[dr], add=True)

        def outer(g, _):
            r0 = base_row + g * INNER
            pltpu.sync_copy(ei_hbm.at[pl.ds(r0, INNER), :, :], ev)
            start_g(slots[0], 0)

            def pipe(jj, _2):
                start_g(slots[1], 2 * jj + 1)
                finish(slots[0], 2 * jj)

                @pl.when(jj < INNER // 2 - 1)
                def _():
                    start_g(slots[0], 2 * jj + 2)

                finish(slots[1], 2 * jj + 1)
                return 0

            lax.fori_loop(0, INNER // 2, pipe, 0)
            return 0

        lax.fori_loop(0, OUTER, outer, 0)
        plsc.subcore_barrier()
        pltpu.sync_copy(acc_s.at[pl.ds(sl0, SL), :],
                        s_out.at[c, pl.ds(sl0, SL), :])
        if with_deg:
            doff = pl.multiple_of(c * NP + s * SL, 128)
            pltpu.sync_copy(acc_deg.at[pl.ds(sl0, SL)],
                            deg_out.at[pl.ds(doff, SL)])

    return pl.kernel(body, out_type=tuple(out_type), mesh=mesh,
                     compiler_params=pltpu.CompilerParams(
                         use_tc_tiling_on_sc=False),
                     scratch_types=tuple(scratch))


def _sc_gcn_body(ei_hbm, ytab_hbm, g_out,
                 ev, xrowA, xrowB, z2d, gsemA, gsemB, acc_g):
    c = lax.axis_index("c")
    s = lax.axis_index("s")
    wid = c * 16 + s
    sl0 = pl.multiple_of(s * SL, 128)
    zero16 = jnp.zeros((16,), jnp.float32)
    for t in range(128):
        z2d[pl.ds(t, 1), :] = zero16.reshape(1, 16)
    for t in range(25):
        pltpu.sync_copy(z2d, acc_g.at[pl.ds(sl0 + t * 128, 128), :])
    plsc.subcore_barrier()

    base_row = wid * RW
    slots = ((xrowA, gsemA), (xrowB, gsemB))

    def start_g(slot, j):
        xrow, gsem = slot
        pltpu.async_copy(ytab_hbm.at[ev.at[j, 0]], xrow, gsem)

    def finish(slot, j):
        xrow, gsem = slot
        pltpu.make_async_copy(ytab_hbm.at[ev.at[j, 0]], xrow, gsem).wait()
        pltpu.sync_copy(xrow, acc_g.at[ev.at[j, 1]], add=True)

    def outer(g, _):
        r0 = base_row + g * INNER
        pltpu.sync_copy(ei_hbm.at[pl.ds(r0, INNER), :, :], ev)
        start_g(slots[0], 0)

        def pipe(jj, _2):
            start_g(slots[1], 2 * jj + 1)
            finish(slots[0], 2 * jj)

            @pl.when(jj < INNER // 2 - 1)
            def _():
                start_g(slots[0], 2 * jj + 2)

            finish(slots[1], 2 * jj + 1)
            return 0

        lax.fori_loop(0, INNER // 2, pipe, 0)
        return 0

    lax.fori_loop(0, OUTER, outer, 0)
    plsc.subcore_barrier()
    pltpu.sync_copy(acc_g.at[pl.ds(sl0, SL), :],
                    g_out.at[c, pl.ds(sl0, SL), :])


def _make_sc_gcn():
    mesh = plsc.VectorSubcoreMesh(core_axis_name="c", subcore_axis_name="s")
    return pl.kernel(
        _sc_gcn_body,
        out_type=jax.ShapeDtypeStruct((2, NP, 16), jnp.float32),
        mesh=mesh,
        compiler_params=pltpu.CompilerParams(use_tc_tiling_on_sc=False),
        scratch_types=(
            pltpu.VMEM((INNER, 2, 128), jnp.int32),
            pltpu.VMEM((128, 16), jnp.float32),
            pltpu.VMEM((128, 16), jnp.float32),
            pltpu.VMEM((128, 16), jnp.float32),
            pltpu.SemaphoreType.DMA,
            pltpu.SemaphoreType.DMA,
            pltpu.VMEM_SHARED((NP, 16), jnp.float32),
        ),
    )


def _tc1_body(x_ref, wext_ref, asd_ref, o0, o1, o2, aux_ref, mx_ref):
    xb = x_ref[...]
    xp48 = jax.lax.dot_general(xb, wext_ref[...], (((1,), (0,)), ((), ())),
                               preferred_element_type=jnp.float32)
    lane = jax.lax.broadcasted_iota(jnp.int32, (BN_BLK, 16), 1)
    cols = []
    for h, oref in enumerate((o0, o1, o2)):
        blk = xp48[:, 16 * h:16 * h + 16]
        oref[...] = jnp.where(lane == 15, 1.0, blk)
        a_s = jnp.sum(blk * asd_ref[0:1, 16 * h:16 * h + 16], axis=1,
                      keepdims=True)
        a_d = jnp.sum(blk * asd_ref[1:2, 16 * h:16 * h + 16], axis=1,
                      keepdims=True)
        cols.append((a_s, a_d))
    zpad = jnp.zeros((BN_BLK, 5), jnp.float32)
    auxb = jnp.concatenate(
        [cols[0][1], cols[1][1], cols[2][1], zpad,
         cols[0][0], cols[1][0], cols[2][0], zpad], axis=1)
    aux_ref[...] = auxb
    mx = jnp.max(auxb, axis=0, keepdims=True)
    mx_ref[...] = jnp.broadcast_to(mx, (8, 16))


def _tc2a_body(s0_ref, s1_ref, s2_ref, x0_ref, x1_ref, x2_ref, aux_ref,
               asrow_ref, b1p_ref, whp_ref, bhp_ref, hw_ref, st_ref):
    aux = aux_ref[...]
    hs = []
    for h, (s_ref, x_ref) in enumerate(((s0_ref, x0_ref), (s1_ref, x1_ref),
                                        (s2_ref, x2_ref))):
        ssum = s_ref[0] + s_ref[1]
        a_s = aux[:, 8 + h:9 + h]
        a_d = aux[:, h:h + 1]
        z = a_s + a_d
        al = jnp.maximum(z, 0.2 * z)
        zm = asrow_ref[0, 8 + h] + a_d
        m = jnp.maximum(zm, 0.2 * zm)
        ps = jnp.exp(al - m)
        numer = ssum + ps * x_ref[...]
        denom = ssum[:, 15:16] + ps + 1e-16
        hs.append(numer / denom)
    hcat = jnp.concatenate(hs, axis=1) + b1p_ref[0:1, :]
    hw = jax.lax.dot_general(hcat, whp_ref[...], (((1,), (0,)), ((), ())),
                             preferred_element_type=jnp.float32)
    hw = hw + bhp_ref[0:1, :]
    hw_ref[...] = hw

    @pl.when(pl.program_id(0) == 0)
    def _():
        st_ref[...] = jnp.zeros((8, 16), jnp.float32)

    s1 = jnp.sum(hw, axis=0, keepdims=True)
    s2 = jnp.sum(hw * hw, axis=0, keepdims=True)
    upd = jnp.concatenate([s1, s2, jnp.zeros((6, 16), jnp.float32)], axis=0)
    st_ref[...] = st_ref[...] + upd


def _tc2b_body(hw_ref, dinvb_ref, w2p_ref, m1_ref, isg1_ref, be1_ref,
               c2_ref, y_ref):
    hn = (hw_ref[...] - m1_ref[0:1, :]) * isg1_ref[0:1, :] + be1_ref[0:1, :]
    t = jax.lax.dot_general(hn, w2p_ref[...], (((1,), (0,)), ((), ())),
                            preferred_element_type=jnp.float32)
    y_ref[...] = (t + c2_ref[0:1, :]) * dinvb_ref[...]


def _tc3a_body(g_ref, y_ref, b2p_ref, h2_ref, st_ref):
    y = y_ref[...]
    gsum = g_ref[0] + g_ref[1]
    dinv = y[:, 9:10]
    t = dinv * (gsum + y) + b2p_ref[0:1, :]
    h2_ref[...] = t

    @pl.when(pl.program_id(0) == 0)
    def _():
        st_ref[...] = jnp.zeros((8, 16), jnp.float32)

    s1 = jnp.sum(t, axis=0, keepdims=True)
    s2 = jnp.sum(t * t, axis=0, keepdims=True)
    upd = jnp.concatenate([s1, s2, jnp.zeros((6, 16), jnp.float32)], axis=0)
    st_ref[...] = st_ref[...] + upd


def _tc3b_body(h2_ref, m2_ref, isg2_ref, be2_ref, a1_ref, bb1_ref, a2_ref,
               bl2_ref, z2_ref):
    hn = (h2_ref[...] - m2_ref[0:1, :]) * isg2_ref[0:1, :] + be2_ref[0:1, :]
    z1 = jax.lax.dot_general(hn, a1_ref[...], (((1,), (0,)), ((), ())),
                             preferred_element_type=jnp.float32)
    z1 = z1 + bb1_ref[0:1, :]
    z1 = jnp.maximum(z1, 0.01 * z1)
    z2 = jax.lax.dot_general(z1, a2_ref[...], (((1,), (0,)), ((), ())),
                             preferred_element_type=jnp.float32)
    z2 = z2 + bl2_ref[0:1, :]
    z2_ref[...] = jnp.maximum(z2, 0.01 * z2)


def _tc3c_body(hr_ref, wt_ref, btp_ref, wmu_ref, bmup_ref, wls_ref, blsp_ref,
               mu_ref, ls_ref):
    z3 = jax.lax.dot_general(hr_ref[...], wt_ref[...], (((1,), (0,)), ((), ())),
                             preferred_element_type=jnp.float32)
    z3 = z3 + btp_ref[0:1, :]
    z3 = jnp.maximum(z3, 0.01 * z3)
    mu_ref[...] = jax.lax.dot_general(
        z3, wmu_ref[...], (((1,), (0,)), ((), ())),
        preferred_element_type=jnp.float32) + bmup_ref[0:1, :]
    ls_ref[...] = jax.lax.dot_general(
        z3, wls_ref[...], (((1,), (0,)), ((), ())),
        preferred_element_type=jnp.float32) + blsp_ref[0:1, :]


def _row8(v, width):
    out = jnp.zeros((8, width), jnp.float32)
    return out.at[0, :v.shape[0]].set(v)


def kernel(x, edge_index, W1, a_src, a_dst, b1, Wh, bh, g1, be1, W2, b2, g2,
           be2, Wl1, bl1, Wl2, bl2, Wt, bt, Wmu, bmu, Wls, bls):
    f32 = jnp.float32
    src = edge_index[0]
    dst = edge_index[1]
    padn = EP - E
    srcp = jnp.concatenate([src, jnp.full((padn,), N, jnp.int32)])
    dstp = jnp.concatenate([dst, jnp.full((padn,), N, jnp.int32)])
    ei2 = jnp.stack([srcp.reshape(NROWS, 128),
                     dstp.reshape(NROWS, 128)], axis=1)

    W1r = W1.reshape(IN, H, C1)
    Wext = jnp.concatenate([W1r, jnp.zeros((IN, H, 1), f32)],
                           axis=2).reshape(IN, 48)
    apad = jnp.zeros((H, 1), f32)
    asd = jnp.zeros((8, 48), f32)
    asd = asd.at[0].set(jnp.concatenate([a_src, apad], 1).reshape(48))
    asd = asd.at[1].set(jnp.concatenate([a_dst, apad], 1).reshape(48))

    grid25 = (NBLK,)
    bs_x = pl.BlockSpec((BN_BLK, IN), lambda i: (i, 0))
    bs_full = lambda shape: pl.BlockSpec(shape, lambda i: tuple(0 for _ in shape))
    bs_n16 = pl.BlockSpec((BN_BLK, 16), lambda i: (i, 0))
    xpad0, xpad1, xpad2, aux, mx = pl.pallas_call(
        _tc1_body,
        grid=grid25,
        in_specs=[bs_x, bs_full((IN, 48)), bs_full((8, 48))],
        out_specs=[bs_n16, bs_n16, bs_n16, bs_n16,
                   pl.BlockSpec((8, 16), lambda i: (i, 0))],
        out_shape=[jax.ShapeDtypeStruct((N, 16), f32)] * 4 +
                  [jax.ShapeDtypeStruct((8 * NBLK, 16), f32)],
    )(x, Wext, asd)

    As3 = jnp.max(mx[:, 8:11], axis=0)
    as16 = [jnp.full((16,), As3[h], f32) for h in range(H)]

    rowpad = jnp.zeros((NP - N, 16), f32)
    wordpad = jnp.zeros((NP - N,), f32)
    xtabs = [jnp.concatenate([xp, rowpad], axis=0)
             for xp in (xpad0, xpad1, xpad2)]
    sas = [jnp.concatenate([aux[:, 8 + h], wordpad]) for h in range(H)]
    das = [jnp.concatenate([aux[:, h], wordpad]) for h in range(H)]

    head0 = _make_sc_head(True)
    headx = _make_sc_head(False)
    S0, degp = head0(ei2, xtabs[0], sas[0], das[0], as16[0])
    (S1,) = headx(ei2, xtabs[1], sas[1], das[1], as16[1])
    (S2,) = headx(ei2, xtabs[2], sas[2], das[2], as16[2])

    degp = degp.reshape(2, NP)
    deg = degp[0, :N] + degp[1, :N] + 1.0
    dinv = 1.0 / jnp.sqrt(jnp.maximum(deg, 1e-12))
    dinvb = jnp.broadcast_to(dinv[:, None], (N, 16))

    Whr = Wh.reshape(H, C1, C1)
    Whp = jnp.concatenate([Whr, jnp.zeros((H, 1, C1), f32)],
                          axis=1).reshape(48, C1)
    Whp = jnp.concatenate([Whp, jnp.zeros((48, 1), f32)], axis=1)
    b1r = b1.reshape(H, C1)
    b1p48 = jnp.concatenate([b1r, jnp.zeros((H, 1), f32)], axis=1).reshape(48)
    asrow = _row8(jnp.concatenate([jnp.zeros((8,), f32), As3]), 16)

    bs_s = pl.BlockSpec((2, BN_BLK, 16), lambda i: (0, i, 0))
    hw, st1 = pl.pallas_call(
        _tc2a_body,
        grid=grid25,
        in_specs=[bs_s, bs_s, bs_s, bs_n16, bs_n16, bs_n16, bs_n16,
                  bs_full((8, 16)), bs_full((8, 48)), bs_full((48, 16)),
                  bs_full((8, 16))],
        out_specs=[bs_n16, pl.BlockSpec((8, 16), lambda i: (0, 0))],
        out_shape=[jax.ShapeDtypeStruct((N, 16), f32),
                   jax.ShapeDtypeStruct((8, 16), f32)],
    )(S0, S1, S2, xpad0, xpad1, xpad2, aux, asrow,
      _row8(b1p48, 48), Whp, _row8(bh, 16))

    mean1 = st1[0] / N
    var1 = st1[1] / N - mean1 * mean1
    g1p = jnp.concatenate([g1, jnp.zeros((1,), f32)])
    be1p = jnp.concatenate([be1, jnp.zeros((1,), f32)])
    isg1 = g1p / jnp.sqrt(var1 + 1e-5)
    W2p16 = jnp.zeros((16, 16), f32).at[:C1, :C2].set(W2)
    c2 = jnp.zeros((16,), f32).at[9].set(1.0)

    (y,) = pl.pallas_call(
        _tc2b_body,
        grid=grid25,
        in_specs=[bs_n16, bs_n16, bs_full((16, 16)), bs_full((8, 16)),
                  bs_full((8, 16)), bs_full((8, 16)), bs_full((8, 16))],
        out_specs=[bs_n16],
        out_shape=[jax.ShapeDtypeStruct((N, 16), f32)],
    )(hw, dinvb, W2p16, _row8(mean1, 16), _row8(isg1, 16), _row8(be1p, 16),
      _row8(c2, 16))

    ytab = jnp.concatenate([y, rowpad], axis=0)

    gcn = _make_sc_gcn()
    Gp = gcn(ei2, ytab)

    h2, st2 = pl.pallas_call(
        _tc3a_body,
        grid=grid25,
        in_specs=[bs_s, bs_n16, bs_full((8, 16))],
        out_specs=[bs_n16, pl.BlockSpec((8, 16), lambda i: (0, 0))],
        out_shape=[jax.ShapeDtypeStruct((N, 16), f32),
                   jax.ShapeDtypeStruct((8, 16), f32)],
    )(Gp, y, _row8(b2, 16))

    mean2 = st2[0] / N
    var2 = st2[1] / N - mean2 * mean2
    g2p = jnp.zeros((16,), f32).at[:C2].set(g2)
    be2p = jnp.zeros((16,), f32).at[:C2].set(be2)
    isg2 = g2p / jnp.sqrt(var2 + 1e-5)

    A1p = jnp.zeros((16, 128), f32).at[:C2, :L1].set(Wl1)
    bb1 = jnp.zeros((128,), f32).at[:L1].set(bl1)
    A2p = jnp.zeros((128, 8), f32).at[:L1, :L2].set(Wl2)
    bl2p = jnp.zeros((8,), f32).at[:L2].set(bl2)

    (z2,) = pl.pallas_call(
        _tc3b_body,
        grid=grid25,
        in_specs=[bs_n16, bs_full((8, 16)), bs_full((8, 16)),
                  bs_full((8, 16)), bs_full((16, 128)), bs_full((8, 128)),
                  bs_full((128, 8)), bs_full((8, 8))],
        out_specs=[pl.BlockSpec((BN_BLK, 8), lambda i: (i, 0))],
        out_shape=[jax.ShapeDtypeStruct((N, 8), f32)],
    )(h2, _row8(mean2, 16), _row8(isg2, 16), _row8(be2p, 16), A1p,
      _row8(bb1, 128), A2p, _row8(bl2p, 8))

    hresh = z2[:, :L2].reshape(NB_MOL, NA * L2)

    mu, ls = pl.pallas_call(
        _tc3c_body,
        grid=(1,),
        in_specs=[bs_full((NB_MOL, NA * L2)), bs_full((NA * L2, OUT)),
                  bs_full((8, OUT)), bs_full((OUT, OUT)), bs_full((8, OUT)),
                  bs_full((OUT, OUT)), bs_full((8, OUT))],
        out_specs=[bs_full((NB_MOL, OUT)), bs_full((NB_MOL, OUT))],
        out_shape=[jax.ShapeDtypeStruct((NB_MOL, OUT), f32),
                   jax.ShapeDtypeStruct((NB_MOL, OUT), f32)],
    )(hresh, Wt, _row8(bt, OUT), Wmu, _row8(bmu, OUT), Wls, _row8(bls, OUT))

    return (mu, ls, edge_index)
